# baseline (speedup 1.0000x reference)
import functools

import jax
import jax.numpy as jnp
from jax import lax
from jax.experimental import pallas as pl
from jax.experimental.pallas import tpu as pltpu
from jax.experimental.pallas import tpu_sc as plsc

B = 8
CAP = 64
NT = 128
NQ = 64
D = 256
H = 8
HD = D // H
SIM_T = 0.7
ALPHA = 0.5

CS = 32
NCH = CAP // CS
NSUB = 8


def _select_body(bank_ref, feat_ref, lu_ref, sel_ref, ne_ref):
    bank = bank_ref[0]
    feat = feat_ref[0]
    pooled = jnp.mean(bank, axis=1)
    agg = jnp.mean(feat, axis=0, keepdims=True)
    an = jnp.sqrt(jnp.sum(agg * agg))
    pn = jnp.sqrt(jnp.sum(pooled * pooled, axis=1, keepdims=True))
    dots = jnp.sum(pooled * agg, axis=1, keepdims=True)
    sims = dots / (jnp.maximum(pn, 1e-12) * jnp.maximum(an, 1e-12))

    m = jnp.max(sims)
    ids = lax.broadcasted_iota(jnp.int32, (CAP, 1), 0)
    ms_idx = jnp.min(jnp.where(sims == m, ids, CAP))

    lu = lu_ref[0]
    lmin = jnp.min(lu)
    ids2 = lax.broadcasted_iota(jnp.int32, (1, CAP), 1)
    lru_idx = jnp.min(jnp.where(lu == lmin, ids2, CAP))
    max_lu = jnp.max(lu)

    merge = m > SIM_T
    tgt = jnp.where(merge, ms_idx, lru_idx)

    existing = bank_ref[0, pl.ds(tgt, 1)][0]
    ne_ref[0] = jnp.where(merge, ALPHA * existing + (1.0 - ALPHA) * feat, feat)

    lane = lax.broadcasted_iota(jnp.int32, (1, 1, 8), 2)
    sel_ref[...] = jnp.where(lane == 0, tgt, jnp.where(lane == 1, max_lu, 0))


def _select_call(memory_bank, features, lu3):
    return pl.pallas_call(
        _select_body,
        grid=(B,),
        in_specs=[
            pl.BlockSpec((1, CAP, NT, D), lambda b: (b, 0, 0, 0)),
            pl.BlockSpec((1, NT, D), lambda b: (b, 0, 0)),
            pl.BlockSpec((1, 1, CAP), lambda b: (b, 0, 0)),
        ],
        out_specs=[
            pl.BlockSpec((1, 1, 8), lambda b: (b, 0, 0)),
            pl.BlockSpec((1, NT, D), lambda b: (b, 0, 0)),
        ],
        out_shape=[
            jax.ShapeDtypeStruct((B, 1, 8), jnp.int32),
            jax.ShapeDtypeStruct((B, NT, D), jnp.float32),
        ],
    )(memory_bank, features, lu3)


def _lu_body(sel_hbm, lu_hbm, out_hbm, sel_s, lu_s, sem):
    cid = lax.axis_index("core")
    half = B // 2
    pltpu.async_copy(sel_hbm, sel_s, sem).wait()
    pltpu.async_copy(lu_hbm.at[pl.ds(cid * half, half)], lu_s, sem).wait()

    def body(i, acc):
        return jnp.maximum(acc, sel_s[i, 1])

    step = lax.fori_loop(1, B, body, sel_s[0, 1]) + 1

    @pl.loop(0, half)
    def _(i):
        t = sel_s[cid * half + i, 0]
        lu_s[i, t] = step

    pltpu.async_copy(lu_s, out_hbm.at[pl.ds(cid * half, half)], sem).wait()


def _lu_call(sel2, last_used):
    kern = pl.kernel(
        _lu_body,
        out_type=jax.ShapeDtypeStruct((B, CAP), jnp.int32),
        mesh=plsc.ScalarSubcoreMesh(axis_name="core", num_cores=2),
        scratch_types=[
            pltpu.SMEM((B, 8), jnp.int32),
            pltpu.SMEM((B // 2, CAP), jnp.int32),
            pltpu.SemaphoreType.DMA,
        ],
    )
    return kern(sel2, last_used)


def _attn_body(sel_ref, bank_ref, ne_ref, q_in_ref,
               wq_ref, wk_ref, wv_ref, wo_ref,
               bq_ref, bk_ref, bv_ref, bo_ref,
               obank_ref, out_ref,
               qk_scr, acc_scr, m_scr, l_scr):
    b = pl.program_id(0)
    s = pl.program_id(1)
    tgt = sel_ref[b, 0, 0]

    x = bank_ref[0]
    mask = (lax.broadcasted_iota(jnp.int32, (CS, 1, 1), 0) + s * CS) == tgt
    xn = jnp.where(mask, ne_ref[...], x)
    obank_ref[0] = xn

    x2 = xn.reshape(CS * NT, D)
    x2b = x2.astype(jnp.bfloat16)

    scale = HD ** -0.5

    @pl.when(s == 0)
    def _():
        q = (lax.dot_general(q_in_ref[0], wq_ref[...],
                             (((1,), (1,)), ((), ())),
                             preferred_element_type=jnp.float32) + bq_ref[...])
        wk = wk_ref[...]
        for h in range(H):
            qh = q[:, h * HD:(h + 1) * HD] * scale
            qk_scr[h * NQ:(h + 1) * NQ] = lax.dot_general(
                qh, wk[h * HD:(h + 1) * HD, :],
                (((1,), (0,)), ((), ())),
                preferred_element_type=jnp.float32)

    qk = qk_scr[...]
    half = CS * NT // NSUB
    for j in range(NSUB):
        xj = x2[j * half:(j + 1) * half]
        sc = lax.dot_general(qk, xj, (((1,), (1,)), ((), ())),
                             preferred_element_type=jnp.float32)
        mh = jnp.max(sc, axis=1, keepdims=True)
        pb = jnp.exp(sc - mh).astype(jnp.bfloat16)
        lh = jnp.sum(pb.astype(jnp.float32), axis=1, keepdims=True)
        acc_scr[NSUB * s + j] = lax.dot_general(
            pb, x2b[j * half:(j + 1) * half], (((1,), (0,)), ((), ())),
            preferred_element_type=jnp.float32)
        m_scr[NSUB * s + j] = mh
        l_scr[NSUB * s + j] = lh

    @pl.when(s == NCH - 1)
    def _():
        ms = m_scr[...]
        mx = jnp.max(ms, axis=0)
        w = jnp.exp(ms - mx)
        num = jnp.sum(w * acc_scr[...], axis=0)
        den = jnp.sum(w * l_scr[...], axis=0)
        thn = num / den
        wo = wo_ref[...]
        wv = wv_ref[...]
        fin = bo_ref[...] + lax.dot_general(
            bv_ref[...], wo, (((1,), (1,)), ((), ())),
            preferred_element_type=jnp.float32)
        for h in range(H):
            th = thn[h * NQ:(h + 1) * NQ]
            t2 = lax.dot_general(th, wv[h * HD:(h + 1) * HD, :],
                                 (((1,), (1,)), ((), ())),
                                 preferred_element_type=jnp.float32)
            fin = fin + lax.dot_general(t2, wo[:, h * HD:(h + 1) * HD],
                                        (((1,), (1,)), ((), ())),
                                        preferred_element_type=jnp.float32)
        out_ref[0] = fin


def _attn_call(sel3, memory_bank, new_entry, query, wqt, wkt, wvt, wot,
               bq2, bk2, bv2, bo2):
    return pl.pallas_call(
        _attn_body,
        grid=(B, NCH),
        in_specs=[
            pl.BlockSpec(memory_space=pltpu.SMEM),
            pl.BlockSpec((1, CS, NT, D), lambda b, s: (b, s, 0, 0)),
            pl.BlockSpec((1, NT, D), lambda b, s: (b, 0, 0)),
            pl.BlockSpec((1, NQ, D), lambda b, s: (b, 0, 0)),
            pl.BlockSpec((D, D), lambda b, s: (0, 0)),
            pl.BlockSpec((D, D), lambda b, s: (0, 0)),
            pl.BlockSpec((D, D), lambda b, s: (0, 0)),
            pl.BlockSpec((D, D), lambda b, s: (0, 0)),
            pl.BlockSpec((1, D), lambda b, s: (0, 0)),
            pl.BlockSpec((1, D), lambda b, s: (0, 0)),
            pl.BlockSpec((1, D), lambda b, s: (0, 0)),
            pl.BlockSpec((1, D), lambda b, s: (0, 0)),
        ],
        out_specs=[
            pl.BlockSpec((1, CS, NT, D), lambda b, s: (b, s, 0, 0)),
            pl.BlockSpec((1, NQ, D), lambda b, s: (b, 0, 0)),
        ],
        out_shape=[
            jax.ShapeDtypeStruct((B, CAP, NT, D), jnp.float32),
            jax.ShapeDtypeStruct((B, NQ, D), jnp.float32),
        ],
        scratch_shapes=[
            pltpu.VMEM((H * NQ, D), jnp.float32),
            pltpu.VMEM((NSUB * NCH, H * NQ, D), jnp.float32),
            pltpu.VMEM((NSUB * NCH, H * NQ, 1), jnp.float32),
            pltpu.VMEM((NSUB * NCH, H * NQ, 1), jnp.float32),
        ],
    )(sel3, memory_bank, new_entry, query, wqt, wkt, wvt, wot,
      bq2, bk2, bv2, bo2)


def kernel(features, query, memory_bank, last_used, Wq, bq, Wk, bk, Wv, bv,
           Wo, bo):
    lu3 = last_used.reshape(B, 1, CAP)
    sel3, new_entry = _select_call(memory_bank, features, lu3)
    updated_bank, out = _attn_call(
        sel3, memory_bank, new_entry, query,
        Wq, Wk, Wv, Wo,
        bq.reshape(1, D), bk.reshape(1, D), bv.reshape(1, D),
        bo.reshape(1, D))
    updated_last_used = _lu_call(sel3.reshape(B, 8), last_used)
    return out, updated_bank, updated_last_used


# speedup vs baseline: 1.1089x; 1.1089x over previous
import functools

import jax
import jax.numpy as jnp
from jax import lax
from jax.experimental import pallas as pl
from jax.experimental.pallas import tpu as pltpu
from jax.experimental.pallas import tpu_sc as plsc

B = 8
CAP = 64
NT = 128
NQ = 64
D = 256
H = 8
HD = D // H
SIM_T = 0.7
ALPHA = 0.5

CS = 64
NCH = CAP // CS
NSUB = 8


def _select_body(bank_ref, feat_ref, lu_ref, sel_ref, ne_ref):
    bank = bank_ref[0]
    feat = feat_ref[0]
    pooled = jnp.mean(bank, axis=1)
    agg = jnp.mean(feat, axis=0, keepdims=True)
    an = jnp.sqrt(jnp.sum(agg * agg))
    pn = jnp.sqrt(jnp.sum(pooled * pooled, axis=1, keepdims=True))
    dots = jnp.sum(pooled * agg, axis=1, keepdims=True)
    sims = dots / (jnp.maximum(pn, 1e-12) * jnp.maximum(an, 1e-12))

    m = jnp.max(sims)
    ids = lax.broadcasted_iota(jnp.int32, (CAP, 1), 0)
    ms_idx = jnp.min(jnp.where(sims == m, ids, CAP))

    lu = lu_ref[0]
    lmin = jnp.min(lu)
    ids2 = lax.broadcasted_iota(jnp.int32, (1, CAP), 1)
    lru_idx = jnp.min(jnp.where(lu == lmin, ids2, CAP))
    max_lu = jnp.max(lu)

    merge = m > SIM_T
    tgt = jnp.where(merge, ms_idx, lru_idx)

    existing = bank_ref[0, pl.ds(tgt, 1)][0]
    ne_ref[0] = jnp.where(merge, ALPHA * existing + (1.0 - ALPHA) * feat, feat)

    lane = lax.broadcasted_iota(jnp.int32, (1, 1, 8), 2)
    sel_ref[...] = jnp.where(lane == 0, tgt, jnp.where(lane == 1, max_lu, 0))


def _select_call(memory_bank, features, lu3):
    return pl.pallas_call(
        _select_body,
        grid=(B,),
        in_specs=[
            pl.BlockSpec((1, CAP, NT, D), lambda b: (b, 0, 0, 0)),
            pl.BlockSpec((1, NT, D), lambda b: (b, 0, 0)),
            pl.BlockSpec((1, 1, CAP), lambda b: (b, 0, 0)),
        ],
        out_specs=[
            pl.BlockSpec((1, 1, 8), lambda b: (b, 0, 0)),
            pl.BlockSpec((1, NT, D), lambda b: (b, 0, 0)),
        ],
        out_shape=[
            jax.ShapeDtypeStruct((B, 1, 8), jnp.int32),
            jax.ShapeDtypeStruct((B, NT, D), jnp.float32),
        ],
    )(memory_bank, features, lu3)


def _lu_body(sel_hbm, lu_hbm, out_hbm, sel_s, lu_s, sem):
    cid = lax.axis_index("core")
    half = B // 2
    pltpu.async_copy(sel_hbm, sel_s, sem).wait()
    pltpu.async_copy(lu_hbm.at[pl.ds(cid * half, half)], lu_s, sem).wait()

    def body(i, acc):
        return jnp.maximum(acc, sel_s[i, 1])

    step = lax.fori_loop(1, B, body, sel_s[0, 1]) + 1

    @pl.loop(0, half)
    def _(i):
        t = sel_s[cid * half + i, 0]
        lu_s[i, t] = step

    pltpu.async_copy(lu_s, out_hbm.at[pl.ds(cid * half, half)], sem).wait()


def _lu_call(sel2, last_used):
    kern = pl.kernel(
        _lu_body,
        out_type=jax.ShapeDtypeStruct((B, CAP), jnp.int32),
        mesh=plsc.ScalarSubcoreMesh(axis_name="core", num_cores=2),
        scratch_types=[
            pltpu.SMEM((B, 8), jnp.int32),
            pltpu.SMEM((B // 2, CAP), jnp.int32),
            pltpu.SemaphoreType.DMA,
        ],
    )
    return kern(sel2, last_used)


def _attn_body(sel_ref, bank_ref, ne_ref, q_in_ref,
               wq_ref, wk_ref, wv_ref, wo_ref,
               bq_ref, bk_ref, bv_ref, bo_ref,
               obank_ref, out_ref,
               qk_scr, acc_scr, m_scr, l_scr):
    b = pl.program_id(0)
    s = pl.program_id(1)
    tgt = sel_ref[b, 0, 0]

    x = bank_ref[0]
    mask = (lax.broadcasted_iota(jnp.int32, (CS, 1, 1), 0) + s * CS) == tgt
    xn = jnp.where(mask, ne_ref[...], x)
    obank_ref[0] = xn

    x2 = xn.reshape(CS * NT, D)
    x2b = x2.astype(jnp.bfloat16)

    scale = HD ** -0.5

    @pl.when(s == 0)
    def _():
        q = (lax.dot_general(q_in_ref[0], wq_ref[...],
                             (((1,), (1,)), ((), ())),
                             preferred_element_type=jnp.float32) + bq_ref[...])
        wk = wk_ref[...]
        for h in range(H):
            qh = q[:, h * HD:(h + 1) * HD] * scale
            qk_scr[h * NQ:(h + 1) * NQ] = lax.dot_general(
                qh, wk[h * HD:(h + 1) * HD, :],
                (((1,), (0,)), ((), ())),
                preferred_element_type=jnp.float32)

    qk = qk_scr[...]
    half = CS * NT // NSUB
    for j in range(NSUB):
        xj = x2[j * half:(j + 1) * half]
        sc = lax.dot_general(qk, xj, (((1,), (1,)), ((), ())),
                             preferred_element_type=jnp.float32)
        mh = jnp.max(sc, axis=1, keepdims=True)
        pb = jnp.exp(sc - mh).astype(jnp.bfloat16)
        lh = jnp.sum(pb.astype(jnp.float32), axis=1, keepdims=True)
        acc_scr[NSUB * s + j] = lax.dot_general(
            pb, x2b[j * half:(j + 1) * half], (((1,), (0,)), ((), ())),
            preferred_element_type=jnp.float32)
        m_scr[NSUB * s + j] = mh
        l_scr[NSUB * s + j] = lh

    @pl.when(s == NCH - 1)
    def _():
        ms = m_scr[...]
        mx = jnp.max(ms, axis=0)
        w = jnp.exp(ms - mx)
        num = jnp.sum(w * acc_scr[...], axis=0)
        den = jnp.sum(w * l_scr[...], axis=0)
        thn = num / den
        wo = wo_ref[...]
        wv = wv_ref[...]
        fin = bo_ref[...] + lax.dot_general(
            bv_ref[...], wo, (((1,), (1,)), ((), ())),
            preferred_element_type=jnp.float32)
        for h in range(H):
            th = thn[h * NQ:(h + 1) * NQ]
            t2 = lax.dot_general(th, wv[h * HD:(h + 1) * HD, :],
                                 (((1,), (1,)), ((), ())),
                                 preferred_element_type=jnp.float32)
            fin = fin + lax.dot_general(t2, wo[:, h * HD:(h + 1) * HD],
                                        (((1,), (1,)), ((), ())),
                                        preferred_element_type=jnp.float32)
        out_ref[0] = fin


def _attn_call(sel3, memory_bank, new_entry, query, wqt, wkt, wvt, wot,
               bq2, bk2, bv2, bo2):
    return pl.pallas_call(
        _attn_body,
        grid=(B, NCH),
        in_specs=[
            pl.BlockSpec(memory_space=pltpu.SMEM),
            pl.BlockSpec((1, CS, NT, D), lambda b, s: (b, s, 0, 0)),
            pl.BlockSpec((1, NT, D), lambda b, s: (b, 0, 0)),
            pl.BlockSpec((1, NQ, D), lambda b, s: (b, 0, 0)),
            pl.BlockSpec((D, D), lambda b, s: (0, 0)),
            pl.BlockSpec((D, D), lambda b, s: (0, 0)),
            pl.BlockSpec((D, D), lambda b, s: (0, 0)),
            pl.BlockSpec((D, D), lambda b, s: (0, 0)),
            pl.BlockSpec((1, D), lambda b, s: (0, 0)),
            pl.BlockSpec((1, D), lambda b, s: (0, 0)),
            pl.BlockSpec((1, D), lambda b, s: (0, 0)),
            pl.BlockSpec((1, D), lambda b, s: (0, 0)),
        ],
        out_specs=[
            pl.BlockSpec((1, CS, NT, D), lambda b, s: (b, s, 0, 0)),
            pl.BlockSpec((1, NQ, D), lambda b, s: (b, 0, 0)),
        ],
        out_shape=[
            jax.ShapeDtypeStruct((B, CAP, NT, D), jnp.float32),
            jax.ShapeDtypeStruct((B, NQ, D), jnp.float32),
        ],
        scratch_shapes=[
            pltpu.VMEM((H * NQ, D), jnp.float32),
            pltpu.VMEM((NSUB * NCH, H * NQ, D), jnp.float32),
            pltpu.VMEM((NSUB * NCH, H * NQ, 1), jnp.float32),
            pltpu.VMEM((NSUB * NCH, H * NQ, 1), jnp.float32),
        ],
    )(sel3, memory_bank, new_entry, query, wqt, wkt, wvt, wot,
      bq2, bk2, bv2, bo2)


def kernel(features, query, memory_bank, last_used, Wq, bq, Wk, bk, Wv, bv,
           Wo, bo):
    lu3 = last_used.reshape(B, 1, CAP)
    sel3, new_entry = _select_call(memory_bank, features, lu3)
    updated_bank, out = _attn_call(
        sel3, memory_bank, new_entry, query,
        Wq, Wk, Wv, Wo,
        bq.reshape(1, D), bk.reshape(1, D), bv.reshape(1, D),
        bo.reshape(1, D))
    updated_last_used = _lu_call(sel3.reshape(B, 8), last_used)
    return out, updated_bank, updated_last_used
